# baseline (speedup 1.0000x reference)
import functools

import jax
import jax.numpy as jnp
from jax import lax
from jax.experimental import pallas as pl
from jax.experimental.pallas import tpu as pltpu
from jax.experimental.pallas import tpu_sc as plsc

NC = 2
NS = 16
NW = NC * NS
BPW = 128
NBUF = 10
WBLK = 8192


def _tc_transpose(V, D):
    def body(i_ref, e_ref, o_ref):
        o_ref[...] = lax.dot_general(
            i_ref[...], e_ref[...],
            dimension_numbers=(((0,), (0,)), ((), ())),
            precision=lax.Precision.HIGHEST,
            preferred_element_type=jnp.float32)

    return pl.pallas_call(
        body,
        grid=(pl.cdiv(V, WBLK),),
        in_specs=[pl.BlockSpec((D, WBLK), lambda i: (0, i)),
                  pl.BlockSpec((D, D), lambda i: (0, 0))],
        out_specs=pl.BlockSpec((WBLK, D), lambda i: (i, 0)),
        out_shape=jax.ShapeDtypeStruct((V, D), jnp.float32),
    )


def _sc_gather(B, D, seq):
    b_per_w = BPW * seq
    niter = seq // NBUF
    mesh = plsc.VectorSubcoreMesh(
        core_axis_name="c", subcore_axis_name="s",
        num_cores=NC, num_subcores=NS)

    @functools.partial(
        pl.kernel,
        out_type=jax.ShapeDtypeStruct((B, D), jnp.float32),
        mesh=mesh,
        scratch_types=[
            pltpu.VMEM((seq, BPW), jnp.int32),
            pltpu.VMEM((seq, BPW), jnp.int32),
            pltpu.VMEM((NBUF, BPW, D), jnp.float32),
        ] + [pltpu.SemaphoreType.DMA] * (2 * NBUF),
        compiler_params=pltpu.CompilerParams(use_tc_tiling_on_sc=False),
    )
    def k(table, idx, out, stg, oidx, buf, *sems):
        gsem = sems[:NBUF]
        ssem = sems[NBUF:]
        wid = lax.axis_index("s") * NC + lax.axis_index("c")
        base = wid * b_per_w
        lane_seq = lax.iota(jnp.int32, 16) * seq

        pltpu.sync_copy(idx.at[:, pl.ds(wid * BPW, BPW)], stg)

        def oidx_body(s, carry):
            for ib in range(BPW // 16):
                oidx[s, pl.ds(ib * 16, 16)] = (
                    lane_seq + (base + ib * 16 * seq + s))
            return carry
        lax.fori_loop(0, seq, oidx_body, 0)

        def issue_chunk(s, b):
            pltpu.async_copy(table.at[stg.at[s]], buf.at[b], gsem[b])

        def wait_gather(b):
            pltpu.make_async_copy(
                table.at[pl.ds(0, BPW)], buf.at[b], gsem[b]).wait()

        def wait_scatter(b):
            pltpu.make_async_copy(
                buf.at[b], out.at[pl.ds(0, BPW)], ssem[b]).wait()

        for b in range(NBUF):
            issue_chunk(b, b)

        def body(i, carry):
            for b in range(NBUF):
                s = i * NBUF + b
                wait_gather(b)
                pltpu.async_copy(buf.at[b], out.at[oidx.at[s]], ssem[b])

                @pl.when(i < niter - 1)
                def _():
                    wait_scatter(b)
                    issue_chunk(s + NBUF, b)
            return carry
        lax.fori_loop(0, niter, body, 0)

        for b in range(NBUF):
            wait_scatter(b)

    return k


def kernel(src_table, trg_table, src_indices, trg_indices):
    batch, seq = src_indices.shape
    V, D = src_table.shape
    B = batch * seq
    tr = _tc_transpose(V, D)
    g = _sc_gather(B, D, seq)
    eye = jnp.eye(D, dtype=jnp.float32)
    out_t = g(trg_table, trg_indices.T.astype(jnp.int32))
    out_s = g(tr(src_table.T, eye), src_indices.T.astype(jnp.int32))
    return (out_s.reshape(batch, seq, D), out_t.reshape(batch, seq, D))


# speedup vs baseline: 1.1294x; 1.1294x over previous
import functools

import jax
import jax.numpy as jnp
from jax import lax
from jax.experimental import pallas as pl
from jax.experimental.pallas import tpu as pltpu
from jax.experimental.pallas import tpu_sc as plsc

NC = 2
NS = 16
NW = NC * NS
BPW = 128
NBUF = 10


def _build(B, D, seq):
    b_per_w = BPW * seq
    niter = seq // NBUF
    mesh = plsc.VectorSubcoreMesh(
        core_axis_name="c", subcore_axis_name="s",
        num_cores=NC, num_subcores=NS)

    @functools.partial(
        pl.kernel,
        out_type=(jax.ShapeDtypeStruct((B, D), jnp.float32),
                  jax.ShapeDtypeStruct((B, D), jnp.float32)),
        mesh=mesh,
        scratch_types=[
            pltpu.VMEM((seq, BPW), jnp.int32),
            pltpu.VMEM((seq, BPW), jnp.int32),
            pltpu.VMEM((seq, BPW), jnp.int32),
            pltpu.VMEM((NBUF, BPW, D), jnp.float32),
        ] + [pltpu.SemaphoreType.DMA] * (2 * NBUF),
        compiler_params=pltpu.CompilerParams(use_tc_tiling_on_sc=False),
    )
    def k(src_t, trg_t, sidx, tidx, out_s, out_t, stg_s, stg_t, oidx, buf,
          *sems):
        gsem = sems[:NBUF]
        ssem = sems[NBUF:]
        wid = lax.axis_index("s") * NC + lax.axis_index("c")
        base = wid * b_per_w
        lane_seq = lax.iota(jnp.int32, 16) * seq

        pltpu.sync_copy(sidx.at[:, pl.ds(wid * BPW, BPW)], stg_s)
        pltpu.sync_copy(tidx.at[:, pl.ds(wid * BPW, BPW)], stg_t)

        def oidx_body(s, carry):
            for ib in range(BPW // 16):
                oidx[s, pl.ds(ib * 16, 16)] = (
                    lane_seq + (base + ib * 16 * seq + s))
            return carry
        lax.fori_loop(0, seq, oidx_body, 0)

        def issue_chunk(table, stg, s, b):
            pltpu.async_copy(table.at[stg.at[s]], buf.at[b], gsem[b])

        def wait_gather(table, b):
            pltpu.make_async_copy(
                table.at[pl.ds(0, BPW)], buf.at[b], gsem[b]).wait()

        def wait_scatter(out, b):
            pltpu.make_async_copy(
                buf.at[b], out.at[pl.ds(0, BPW)], ssem[b]).wait()

        def run_table(table, stg, out, drain_prev):
            for b in range(NBUF):
                if drain_prev:
                    wait_scatter(out, b)
                issue_chunk(table, stg, b, b)

            def body(i, carry):
                for b in range(NBUF):
                    s = i * NBUF + b
                    wait_gather(table, b)
                    pltpu.async_copy(buf.at[b], out.at[oidx.at[s]], ssem[b])

                    @pl.when(i < niter - 1)
                    def _():
                        wait_scatter(out, b)
                        issue_chunk(table, stg, s + NBUF, b)
                return carry
            lax.fori_loop(0, niter, body, 0)

        run_table(src_t, stg_s, out_s, False)
        run_table(trg_t, stg_t, out_t, True)
        for b in range(NBUF):
            wait_scatter(out_t, b)

    return k


def kernel(src_table, trg_table, src_indices, trg_indices):
    batch, seq = src_indices.shape
    D = src_table.shape[1]
    B = batch * seq
    sidx = src_indices.T.astype(jnp.int32)
    tidx = trg_indices.T.astype(jnp.int32)
    out_s, out_t = _build(B, D, seq)(src_table, trg_table, sidx, tidx)
    return (out_s.reshape(batch, seq, D), out_t.reshape(batch, seq, D))
